# Initial kernel scaffold; baseline (speedup 1.0000x reference)
import jax
import jax.numpy as jnp
from jax.experimental import pallas as pl


def kernel(x, batch, W1, b1, W2, b2):
    raise NotImplementedError("write your pallas kernel here")


# speedup vs baseline: 1581.8903x; 1581.8903x over previous
import functools

import jax
import jax.numpy as jnp
from jax.experimental import pallas as pl

N = 1024
G = 8


def _dot(a, b):
    return jax.lax.dot_general(
        a, b, (((1,), (0,)), ((), ())),
        preferred_element_type=jnp.float32,
        precision=jax.lax.Precision.HIGHEST,
    )


def _dot_t(a, b):
    return jax.lax.dot_general(
        a, b, (((0,), (0,)), ((), ())),
        preferred_element_type=jnp.float32,
        precision=jax.lax.Precision.HIGHEST,
    )


def _gnn_body(x_ref, batch_ref, w1_ref, b1_ref, w2_ref, b2_ref, out_ref):
    x = x_ref[...]

    ones_col = jnp.ones((N, 1), jnp.float32)
    deg_col = _dot_t(x, ones_col) + 1.0
    s_col = jax.lax.rsqrt(deg_col)

    q1 = _dot(x, w1_ref[...]) * s_col
    h1 = jax.nn.relu((_dot_t(x, q1) + q1) * s_col + b1_ref[...])

    q2 = _dot(h1, w2_ref[...]) * s_col
    h2 = (_dot_t(x, q2) + q2) * s_col + b2_ref[...]

    seg = jax.lax.broadcasted_iota(jnp.int32, (G, N), 0)
    m = (batch_ref[...] == seg).astype(jnp.float32)
    cnt = jnp.sum(m, axis=1, keepdims=True)
    out_ref[...] = _dot(m, h2) / jnp.maximum(cnt, 1.0)


@functools.partial(jax.jit, static_argnames=())
def kernel(x, batch, W1, b1, W2, b2):
    return pl.pallas_call(
        _gnn_body,
        out_shape=jax.ShapeDtypeStruct((G, N), jnp.float32),
    )(
        x.astype(jnp.float32),
        batch.astype(jnp.int32).reshape(1, N),
        W1.astype(jnp.float32),
        b1.astype(jnp.float32).reshape(1, N // 2),
        W2.astype(jnp.float32),
        b2.astype(jnp.float32).reshape(1, N),
    )


# speedup vs baseline: 2202.2405x; 1.3922x over previous
import functools

import jax
import jax.numpy as jnp
from jax.experimental import pallas as pl

N = 1024
G = 8


def _dot(a, b):
    return jax.lax.dot_general(
        a, b, (((1,), (0,)), ((), ())),
        preferred_element_type=jnp.float32,
        precision=jax.lax.Precision.HIGHEST,
    )


def _dot_t(a, b):
    return jax.lax.dot_general(
        a, b, (((0,), (0,)), ((), ())),
        preferred_element_type=jnp.float32,
        precision=jax.lax.Precision.HIGHEST,
    )


def _gnn_body(x_ref, batch_ref, w1_ref, b1_ref, w2_ref, b2_ref, out_ref):
    x = x_ref[...]

    ones_col = jnp.ones((N, 1), jnp.float32)
    deg_col = _dot_t(x, ones_col) + 1.0
    s_col = jax.lax.rsqrt(deg_col)

    q1 = _dot(x, w1_ref[...]) * s_col
    h1 = jax.nn.relu((_dot_t(x, q1) + q1) * s_col + b1_ref[...])

    q2 = _dot(h1, w2_ref[...]) * s_col
    seg = jax.lax.broadcasted_iota(jnp.int32, (N, G), 1)
    mt = (batch_ref[...] == seg).astype(jnp.float32)
    cnt_col = _dot_t(mt, ones_col)
    wt = mt * s_col
    vt = _dot(x, wt) + wt
    acc = _dot_t(vt, q2) + cnt_col * b2_ref[...]
    out_ref[...] = acc / jnp.maximum(cnt_col, 1.0)


@functools.partial(jax.jit, static_argnames=())
def kernel(x, batch, W1, b1, W2, b2):
    return pl.pallas_call(
        _gnn_body,
        out_shape=jax.ShapeDtypeStruct((G, N), jnp.float32),
    )(
        x.astype(jnp.float32),
        batch.astype(jnp.int32).reshape(N, 1),
        W1.astype(jnp.float32),
        b1.astype(jnp.float32).reshape(1, N // 2),
        W2.astype(jnp.float32),
        b2.astype(jnp.float32).reshape(1, N),
    )


# speedup vs baseline: 7411.3226x; 3.3654x over previous
import functools

import jax
import jax.numpy as jnp
from jax.experimental import pallas as pl

N = 1024
G = 8


def _dot(a, b):
    return jax.lax.dot_general(
        a, b, (((1,), (0,)), ((), ())),
        preferred_element_type=jnp.float32,
        precision=jax.lax.Precision.DEFAULT,
    )


def _dot_t(a, b):
    return jax.lax.dot_general(
        a, b, (((0,), (0,)), ((), ())),
        preferred_element_type=jnp.float32,
        precision=jax.lax.Precision.DEFAULT,
    )


def _gnn_body(x_ref, batch_ref, w1_ref, b1_ref, w2_ref, b2_ref, out_ref):
    x = x_ref[...]

    ones_col = jnp.ones((N, 1), jnp.float32)
    deg_col = _dot_t(x, ones_col) + 1.0
    s_col = jax.lax.rsqrt(deg_col)

    q1 = _dot(x, w1_ref[...]) * s_col
    h1 = jax.nn.relu((_dot_t(x, q1) + q1) * s_col + b1_ref[...])

    q2 = _dot(h1, w2_ref[...]) * s_col
    seg = jax.lax.broadcasted_iota(jnp.int32, (N, G), 1)
    mt = (batch_ref[...] == seg).astype(jnp.float32)
    cnt_col = _dot_t(mt, ones_col)
    wt = mt * s_col
    vt = _dot(x, wt) + wt
    acc = _dot_t(vt, q2) + cnt_col * b2_ref[...]
    out_ref[...] = acc / jnp.maximum(cnt_col, 1.0)


@functools.partial(jax.jit, static_argnames=())
def kernel(x, batch, W1, b1, W2, b2):
    return pl.pallas_call(
        _gnn_body,
        out_shape=jax.ShapeDtypeStruct((G, N), jnp.float32),
    )(
        x.astype(jnp.float32),
        batch.astype(jnp.int32).reshape(N, 1),
        W1.astype(jnp.float32),
        b1.astype(jnp.float32).reshape(1, N // 2),
        W2.astype(jnp.float32),
        b2.astype(jnp.float32).reshape(1, N),
    )
